# baseline (speedup 1.0000x reference)
import functools
import math

import jax
import jax.numpy as jnp
from jax import lax
from jax.experimental import pallas as pl
from jax.experimental.pallas import tpu as pltpu
from jax.experimental.pallas import tpu_sc as plsc

NC = 2
NS = 16
NW = NC * NS
LANES = 16
EMB = 64
SCALE = math.sqrt(EMB)
SPLITS = ((0, 104), (104, 96))
NBUF = 4


@jax.jit
def _lookup(tokens, table):
    n_rows, row_len = tokens.shape
    rows_per_w = n_rows // NW

    mesh = plsc.VectorSubcoreMesh(core_axis_name="c", subcore_axis_name="s")

    row_bufs = [pltpu.VMEM((row_len, EMB), jnp.float32) for _ in range(NBUF)]
    gsems = [pltpu.SemaphoreType.DMA for _ in range(NBUF)]
    ssems = [pltpu.SemaphoreType.DMA for _ in range(NBUF)]

    @functools.partial(
        pl.kernel,
        out_type=jax.ShapeDtypeStruct((n_rows, row_len, EMB), jnp.float32),
        mesh=mesh,
        scratch_types=[pltpu.VMEM((rows_per_w, row_len), jnp.int32)]
        + row_bufs + gsems + ssems,
        compiler_params=pltpu.CompilerParams(use_tc_tiling_on_sc=False),
    )
    def body(tok_hbm, table_hbm, out_hbm, idx_v, *refs):
        rows = refs[:NBUF]
        gsem = refs[NBUF:2 * NBUF]
        ssem = refs[2 * NBUF:3 * NBUF]

        wid = lax.axis_index("s") * NC + lax.axis_index("c")
        row0 = wid * rows_per_w

        pltpu.sync_copy(tok_hbm.at[pl.ds(row0, rows_per_w)], idx_v)

        def gather_descs(r, b):
            return [
                pltpu.make_async_copy(
                    table_hbm.at[idx_v.at[r, pl.ds(off, sz)]],
                    rows[b].at[pl.ds(off, sz)],
                    gsem[b],
                )
                for off, sz in SPLITS
            ]

        def store_desc(r, b):
            return pltpu.make_async_copy(rows[b], out_hbm.at[row0 + r], ssem[b])

        def scale(b):
            @pl.loop(0, row_len, unroll=4)
            def _rows(r):
                for c in range(EMB // LANES):
                    sl = pl.ds(c * LANES, LANES)
                    rows[b][r, sl] = rows[b][r, sl] * SCALE

        for b in range(NBUF):
            for d in gather_descs(b, b):
                d.start()

        @pl.loop(0, rows_per_w, step=NBUF)
        def _ring(g):
            for b in range(NBUF):
                r = g + b
                for d in gather_descs(r, b):
                    d.wait()
                scale(b)
                store_desc(r, b).start()

                @pl.when(r + NBUF < rows_per_w)
                def _():
                    store_desc(r, b).wait()
                    for d in gather_descs(r + NBUF, b):
                        d.start()

        for b in range(NBUF):
            store_desc(rows_per_w - NBUF + b, b).wait()

    return body(tokens, table)


def kernel(tokens, table):
    if tokens.dtype != jnp.int32:
        tokens = tokens.astype(jnp.int32)
    return _lookup(tokens, table)
